# Initial kernel scaffold; baseline (speedup 1.0000x reference)
import jax
import jax.numpy as jnp
from jax.experimental import pallas as pl


def kernel(input):
    raise NotImplementedError("write your pallas kernel here")


# speedup vs baseline: 101.4277x; 101.4277x over previous
import functools

import jax
import jax.numpy as jnp
from jax import lax
from jax.experimental import pallas as pl
from jax.experimental.pallas import tpu as pltpu
from jax.experimental.pallas import tpu_sc as plsc

B = 8
W = 2048
DIST = 10
L = 16
NBLK = W // L
NSUP = NBLK // L
NEG_INF = float("-inf")


def _splat_f(s):
    return lax.broadcast_in_dim(jnp.float32(s), (L,), ())


def _splat_i(s):
    return lax.broadcast_in_dim(jnp.int32(s), (L,), ())


def _row_program(xrow, key, bmax, smax, outrow):
    lanes = lax.iota(jnp.int32, L)
    lane0 = lanes == 0
    neg_inf_v = _splat_f(NEG_INF)

    def detect(b, _):
        base = b * L
        pos = base + lanes
        xc = xrow[pl.ds(base, L)]
        xp = plsc.load_gather(xrow, [jnp.maximum(pos - 1, 0)])
        xn = plsc.load_gather(xrow, [jnp.minimum(pos + 1, W - 1)])
        dxr = (pos < W - 1) & ((xn - xc) > 0)
        dxl = (pos == 0) | ((xc - xp) <= 0)
        nonpos = xc <= 0
        valley = dxr & dxl & nonpos
        peak = (~dxr) & (~dxl) & (~nonpos)
        k = jnp.where(valley | peak, jnp.abs(xc), neg_inf_v)
        key[pl.ds(base, L)] = k
        plsc.store_scatter(bmax, [_splat_i(b)], _splat_f(jnp.max(k)), mask=lane0)
        outrow[pl.ds(base, L)] = jnp.zeros((L,), jnp.float32)
        return 0

    lax.fori_loop(0, NBLK, detect, 0)

    smax[...] = neg_inf_v

    def sup(s, _):
        m = jnp.max(bmax[pl.ds(s * L, L)])
        plsc.store_scatter(smax, [_splat_i(s)], _splat_f(m), mask=lane0)
        return 0

    lax.fori_loop(0, NSUP, sup, 0)

    big = _splat_i(10 * W)

    def cond(m):
        return m > NEG_INF

    def body(m):
        sm = smax[...]
        s_sel = jnp.min(jnp.where(sm == m, lanes, big))
        bm = bmax[pl.ds(s_sel * L, L)]
        b_sel = s_sel * L + jnp.min(jnp.where(bm == m, lanes, big))
        kv = key[pl.ds(b_sel * L, L)]
        p = b_sel * L + jnp.min(jnp.where(kv == m, lanes, big))

        xv = plsc.load_gather(xrow, [_splat_i(p)])
        plsc.store_scatter(outrow, [_splat_i(p)], xv, mask=lane0)

        w1 = p - DIST + lanes
        m1 = (w1 >= 0) & (w1 <= W - 1)
        plsc.store_scatter(key, [jnp.clip(w1, 0, W - 1)], neg_inf_v, mask=m1)
        w2 = p + 6 + lanes
        m2 = (lanes <= DIST - 6) & (w2 <= W - 1)
        plsc.store_scatter(key, [jnp.clip(w2, 0, W - 1)], neg_inf_v, mask=m2)

        lo = jnp.maximum(p - DIST, 0)
        hi = jnp.minimum(p + DIST, W - 1)
        ba = lo // L
        bz = hi // L
        for t in range(3):
            bb = jnp.minimum(ba + t, bz)
            mb = jnp.max(key[pl.ds(bb * L, L)])
            plsc.store_scatter(bmax, [_splat_i(bb)], _splat_f(mb), mask=lane0)
        for t in range(2):
            sb = jnp.minimum(ba // L + t, bz // L)
            ms = jnp.max(bmax[pl.ds(sb * L, L)])
            plsc.store_scatter(smax, [_splat_i(sb)], _splat_f(ms), mask=lane0)

        return jnp.max(smax[...])

    lax.while_loop(cond, body, jnp.max(smax[...]))


_mesh = plsc.VectorSubcoreMesh(
    core_axis_name="c", subcore_axis_name="s", num_cores=2, num_subcores=16
)
_SCRATCH = [
    pltpu.VMEM((W,), jnp.float32),
    pltpu.VMEM((W,), jnp.float32),
    pltpu.VMEM((NBLK,), jnp.float32),
    pltpu.VMEM((L,), jnp.float32),
    pltpu.VMEM((W,), jnp.float32),
]


def _extrema_nms_body(x_hbm, out_hbm, xrow, key, bmax, smax, outrow):
    wid = lax.axis_index("s") * 2 + lax.axis_index("c")

    @pl.when(wid < B)
    def _():
        pltpu.sync_copy(x_hbm.at[wid], xrow)
        _row_program(xrow, key, bmax, smax, outrow)
        pltpu.sync_copy(outrow, out_hbm.at[wid])


_extrema_nms = pl.kernel(
    _extrema_nms_body,
    out_type=jax.ShapeDtypeStruct((B, W), jnp.float32),
    mesh=_mesh,
    scratch_types=_SCRATCH,
    compiler_params=pltpu.CompilerParams(needs_layout_passes=False),
)


@jax.jit
def kernel(input):
    x = input.reshape(B, W)
    out = _extrema_nms(x)
    return out.reshape(B, 1, W)


# speedup vs baseline: 111.8106x; 1.1024x over previous
import jax
import jax.numpy as jnp
from jax import lax
from jax.experimental import pallas as pl
from jax.experimental.pallas import tpu as pltpu
from jax.experimental.pallas import tpu_sc as plsc

B = 8
W = 2048
DIST = 10
L = 16
NBLK = W // L
NGRP = NBLK // L
MAX_PICKS = (W - 1) // (DIST + 1) + 1
NEG_INF = float("-inf")


def _splat_f(s):
    return lax.broadcast_in_dim(jnp.float32(s), (L,), ())


def _splat_i(s):
    return lax.broadcast_in_dim(jnp.int32(s), (L,), ())


def _bcast_last(v):
    return jnp.take_along_axis(v, _splat_i(L - 1), axis=0)


def _vmax_splat(v):
    return _bcast_last(plsc.cummax(v))


def _tree(op, xs):
    while len(xs) > 1:
        xs = [op(xs[i], xs[i + 1]) for i in range(0, len(xs) - 1, 2)] + (
            [xs[-1]] if len(xs) % 2 else []
        )
    return xs[0]


def _row_program(xrow, key, bmax, outrow):
    lanes = lax.iota(jnp.int32, L)
    lane0 = lanes == 0
    neg_inf_v = _splat_f(NEG_INF)
    big = _splat_i(32 * W)

    def detect(b, _):
        base = b * L
        pos = base + lanes
        xc = xrow[pl.ds(base, L)]
        xp = plsc.load_gather(xrow, [jnp.maximum(pos - 1, 0)])
        xn = plsc.load_gather(xrow, [jnp.minimum(pos + 1, W - 1)])
        dxr = (pos < W - 1) & ((xn - xc) > 0)
        dxl = (pos == 0) | ((xc - xp) <= 0)
        nonpos = xc <= 0
        valley = dxr & dxl & nonpos
        peak = (~dxr) & (~dxl) & (~nonpos)
        k = jnp.where(valley | peak, jnp.abs(xc), neg_inf_v)
        key[pl.ds(base, L)] = k
        plsc.store_scatter(
            bmax, [lax.broadcast_in_dim(b, (L,), ())], _vmax_splat(k), mask=lane0
        )
        outrow[pl.ds(base, L)] = jnp.zeros((L,), jnp.float32)
        return 0

    lax.fori_loop(0, NBLK, detect, 0, unroll=4)

    def body(_, carry):
        bmv = [bmax[pl.ds(v * L, L)] for v in range(NGRP)]
        m = _vmax_splat(_tree(jnp.maximum, bmv))
        valid = m > neg_inf_v

        sels = []
        for v in range(NGRP):
            ffs = plsc.all_reduce_ffs(bmv[v] == m)
            sels.append(jnp.where(ffs >= L, big, v * L + ffs))
        bsel = _tree(jnp.minimum, sels)

        kv = plsc.load_gather(key, [bsel * L + lanes])
        p = bsel * L + plsc.all_reduce_ffs(kv == m)

        xv = plsc.load_gather(xrow, [p])
        plsc.store_scatter(outrow, [p], xv, mask=lane0 & valid)

        w1 = p - DIST + lanes
        m1 = (w1 >= 0) & (w1 <= W - 1) & valid
        plsc.store_scatter(key, [jnp.clip(w1, 0, W - 1)], neg_inf_v, mask=m1)
        w2 = p + DIST - 4 + lanes
        m2 = (lanes <= 4) & (w2 <= W - 1) & valid
        plsc.store_scatter(key, [jnp.clip(w2, 0, W - 1)], neg_inf_v, mask=m2)

        ba = jnp.maximum(p - DIST, 0) // L
        bz = jnp.minimum(p + DIST, W - 1) // L
        for t in range(3):
            bb = jnp.minimum(ba + t, bz)
            kvb = plsc.load_gather(key, [bb * L + lanes])
            plsc.store_scatter(bmax, [bb], _vmax_splat(kvb), mask=lane0)
        return carry

    lax.fori_loop(0, MAX_PICKS, body, 0)


_mesh = plsc.VectorSubcoreMesh(
    core_axis_name="c", subcore_axis_name="s", num_cores=2, num_subcores=16
)
_SCRATCH = [
    pltpu.VMEM((W,), jnp.float32),
    pltpu.VMEM((W,), jnp.float32),
    pltpu.VMEM((NBLK,), jnp.float32),
    pltpu.VMEM((W,), jnp.float32),
]


def _extrema_nms_body(x_hbm, out_hbm, xrow, key, bmax, outrow):
    wid = lax.axis_index("s") * 2 + lax.axis_index("c")

    @pl.when(wid < B)
    def _():
        pltpu.sync_copy(x_hbm.at[wid], xrow)
        _row_program(xrow, key, bmax, outrow)
        pltpu.sync_copy(outrow, out_hbm.at[wid])


_extrema_nms = pl.kernel(
    _extrema_nms_body,
    out_type=jax.ShapeDtypeStruct((B, W), jnp.float32),
    mesh=_mesh,
    scratch_types=_SCRATCH,
    compiler_params=pltpu.CompilerParams(needs_layout_passes=False),
)


@jax.jit
def kernel(input):
    x = input.reshape(B, W)
    out = _extrema_nms(x)
    return out.reshape(B, 1, W)
